# baseline (speedup 1.0000x reference)
import jax
import jax.numpy as jnp
from jax import lax
from jax.experimental import pallas as pl
from jax.experimental.pallas import tpu as pltpu
from jax.experimental.pallas import tpu_sc as plsc

N_TOKENS = 100000
D_MODEL = 768
MAX_LEN = 8192
BATCH = 4
SEQ = 4096

NC = 2
NS = 16
NW = NC * NS
LANES = 16

S_PER_W = SEQ // NW
CHUNK = 32
N_SBLK = S_PER_W // CHUNK
N_STEP = N_SBLK * BATCH
D_VECS = D_MODEL // LANES


def _emb_kernel(ids_hbm, wemb_hbm, pos_hbm, out_hbm,
                idx_v, pos_v, row_v, gs0, gs1, isem):
    wid = lax.axis_index("s") * NC + lax.axis_index("c")
    s0 = wid * S_PER_W

    idx_copies = []
    for g in range(N_STEP):
        j, b = divmod(g, BATCH)
        idx_copies.append(pltpu.async_copy(
            ids_hbm.at[b, pl.ds(s0 + j * CHUNK, CHUNK)], idx_v.at[g], isem))
    for c in idx_copies:
        c.wait()

    def fire(g, p, sem):
        pltpu.async_copy(wemb_hbm.at[idx_v.at[g]], row_v.at[p], sem)

    def wait(g, p, sem):
        pltpu.make_async_copy(
            wemb_hbm.at[idx_v.at[g]], row_v.at[p], sem).wait()

    fire(0, 0, gs0)

    def step(g, _):
        p = lax.rem(g, 2)
        j = lax.div(g, BATCH)
        b = lax.rem(g, BATCH)

        @pl.when(g + 1 < N_STEP)
        def _():
            pn = lax.rem(g + 1, 2)

            @pl.when(pn == 0)
            def _():
                fire(g + 1, 0, gs0)

            @pl.when(pn == 1)
            def _():
                fire(g + 1, 1, gs1)

        @pl.when(b == 0)
        def _():
            pltpu.sync_copy(pos_hbm.at[pl.ds(s0 + j * CHUNK, CHUNK)], pos_v)

        @pl.when(p == 0)
        def _():
            wait(g, 0, gs0)

        @pl.when(p == 1)
        def _():
            wait(g, 1, gs1)

        def body(r, _):
            for k in range(D_VECS):
                sl = pl.ds(k * LANES, LANES)
                plsc.addupdate(row_v.at[p, r, sl], pos_v[r, sl])
            return 0
        lax.fori_loop(0, CHUNK, body, 0)

        pltpu.sync_copy(
            row_v.at[p],
            out_hbm.at[pl.ds(b * SEQ + s0 + j * CHUNK, CHUNK)])
        return 0

    lax.fori_loop(0, N_STEP, step, 0)


def kernel(ids, word_emb, pos_table):
    ids32 = ids.astype(jnp.int32)
    mesh = plsc.VectorSubcoreMesh(core_axis_name="c", subcore_axis_name="s")
    out = pl.kernel(
        _emb_kernel,
        mesh=mesh,
        out_type=jax.ShapeDtypeStruct((BATCH * SEQ, D_MODEL), jnp.float32),
        scratch_types=[
            pltpu.VMEM((N_STEP, CHUNK), jnp.int32),
            pltpu.VMEM((CHUNK, D_MODEL), jnp.float32),
            pltpu.VMEM((2, CHUNK, D_MODEL), jnp.float32),
        ] + [pltpu.SemaphoreType.DMA] * 3,
    )(ids32, word_emb, pos_table)
    return out.reshape(BATCH, SEQ, D_MODEL)


# speedup vs baseline: 1.4298x; 1.4298x over previous
import jax
import jax.numpy as jnp
from jax import lax
from jax.experimental import pallas as pl
from jax.experimental.pallas import tpu as pltpu
from jax.experimental.pallas import tpu_sc as plsc

N_TOKENS = 100000
D_MODEL = 768
MAX_LEN = 8192
BATCH = 4
SEQ = 4096

NC = 2
NS = 16
NW = NC * NS
LANES = 16

S_PER_W = SEQ // NW
CHUNK = 32
N_SBLK = S_PER_W // CHUNK
N_STEP = N_SBLK * BATCH
NBUF = 3
D_VECS = D_MODEL // LANES


def _emb_kernel(ids_hbm, wemb_hbm, pos_hbm, out_hbm,
                idx_v, pos_v, row_v,
                gs0, gs1, gs2, ss0, isem):
    wid = lax.axis_index("s") * NC + lax.axis_index("c")
    s0 = wid * S_PER_W
    gsem = (gs0, gs1, gs2)

    idx_copies = []
    for g in range(N_STEP):
        j, b = divmod(g, BATCH)
        idx_copies.append(pltpu.async_copy(
            ids_hbm.at[b, pl.ds(s0 + j * CHUNK, CHUNK)], idx_v.at[g], isem))
    for c in idx_copies:
        c.wait()

    gathers = {}
    stores = {}

    def fire_gather(g):
        gathers[g] = pltpu.async_copy(
            wemb_hbm.at[idx_v.at[g]], row_v.at[g % NBUF], gsem[g % NBUF])

    def add_rows(p):
        def body(r, _):
            for k in range(D_VECS):
                sl = pl.ds(k * LANES, LANES)
                plsc.addupdate(row_v.at[p, r, sl], pos_v[r, sl])
            return 0
        lax.fori_loop(0, CHUNK, body, 0)

    fire_gather(0)
    fire_gather(1)
    for g in range(N_STEP):
        j, b = divmod(g, BATCH)
        waited = False
        if b == 0:
            if g >= 1:
                stores[g - 1].wait()
                waited = True
            pltpu.sync_copy(pos_hbm.at[pl.ds(s0 + j * CHUNK, CHUNK)], pos_v)
        gathers[g].wait()
        add_rows(g % NBUF)
        if g >= 1 and not waited:
            stores[g - 1].wait()
        if g + 2 < N_STEP:
            fire_gather(g + 2)
        stores[g] = pltpu.async_copy(
            row_v.at[g % NBUF],
            out_hbm.at[pl.ds(b * SEQ + s0 + j * CHUNK, CHUNK)],
            ss0)
    stores[N_STEP - 1].wait()


def kernel(ids, word_emb, pos_table):
    ids32 = ids.astype(jnp.int32)
    mesh = plsc.VectorSubcoreMesh(core_axis_name="c", subcore_axis_name="s")
    out = pl.kernel(
        _emb_kernel,
        mesh=mesh,
        out_type=jax.ShapeDtypeStruct((BATCH * SEQ, D_MODEL), jnp.float32),
        scratch_types=[
            pltpu.VMEM((N_STEP, CHUNK), jnp.int32),
            pltpu.VMEM((CHUNK, D_MODEL), jnp.float32),
            pltpu.VMEM((NBUF, CHUNK, D_MODEL), jnp.float32),
        ] + [pltpu.SemaphoreType.DMA] * 5,
    )(ids32, word_emb, pos_table)
    return out.reshape(BATCH, SEQ, D_MODEL)


# speedup vs baseline: 1.5557x; 1.0880x over previous
import jax
import jax.numpy as jnp
from jax import lax
from jax.experimental import pallas as pl
from jax.experimental.pallas import tpu as pltpu
from jax.experimental.pallas import tpu_sc as plsc

N_TOKENS = 100000
D_MODEL = 768
MAX_LEN = 8192
BATCH = 4
SEQ = 4096

NC = 2
NS = 16
NW = NC * NS
LANES = 16

S_PER_W = SEQ // NW
CHUNK = 32
N_SBLK = S_PER_W // CHUNK
N_STEP = N_SBLK * BATCH
D_VECS = D_MODEL // LANES


def _emb_kernel(ids_hbm, wemb_hbm, pos_hbm, out_hbm,
                idx_v, pos_v, row_v,
                gs0, gs1, isem):
    wid = lax.axis_index("s") * NC + lax.axis_index("c")
    s0 = wid * S_PER_W
    gsem = (gs0, gs1)

    idx_copies = []
    for g in range(N_STEP):
        j, b = divmod(g, BATCH)
        idx_copies.append(pltpu.async_copy(
            ids_hbm.at[b, pl.ds(s0 + j * CHUNK, CHUNK)], idx_v.at[g], isem))
    for c in idx_copies:
        c.wait()

    gathers = {}

    def fire_gather(g):
        gathers[g] = pltpu.async_copy(
            wemb_hbm.at[idx_v.at[g]], row_v.at[g % 2], gsem[g % 2])

    def add_rows(p):
        def body(r, _):
            for k in range(D_VECS):
                sl = pl.ds(k * LANES, LANES)
                row_v[p, r, sl] = row_v[p, r, sl] + pos_v[r, sl]
            return 0
        lax.fori_loop(0, CHUNK, body, 0)

    fire_gather(0)
    for g in range(N_STEP):
        p = g % 2
        j, b = divmod(g, BATCH)
        if g + 1 < N_STEP:
            fire_gather(g + 1)
        if b == 0:
            pltpu.sync_copy(pos_hbm.at[pl.ds(s0 + j * CHUNK, CHUNK)], pos_v)
        gathers[g].wait()
        add_rows(p)
        pltpu.sync_copy(
            row_v.at[p],
            out_hbm.at[pl.ds(b * SEQ + s0 + j * CHUNK, CHUNK)])


def kernel(ids, word_emb, pos_table):
    ids32 = ids.astype(jnp.int32)
    mesh = plsc.VectorSubcoreMesh(core_axis_name="c", subcore_axis_name="s")
    out = pl.kernel(
        _emb_kernel,
        mesh=mesh,
        out_type=jax.ShapeDtypeStruct((BATCH * SEQ, D_MODEL), jnp.float32),
        scratch_types=[
            pltpu.VMEM((N_STEP, CHUNK), jnp.int32),
            pltpu.VMEM((CHUNK, D_MODEL), jnp.float32),
            pltpu.VMEM((2, CHUNK, D_MODEL), jnp.float32),
        ] + [pltpu.SemaphoreType.DMA] * 3,
    )(ids32, word_emb, pos_table)
    return out.reshape(BATCH, SEQ, D_MODEL)
